# baseline (speedup 1.0000x reference)
import jax
import jax.numpy as jnp
from jax import lax
from jax.experimental import pallas as pl
from jax.experimental.pallas import tpu as pltpu
from jax.experimental.pallas import tpu_sc as plsc

N_FIELDS = 26
N_PAIRS = N_FIELDS * (N_FIELDS - 1) // 2
D = 128
L = 16
NV = D // L
B = 4096
NC = 2
NS = 16
NW = NC * NS
BPW = B // NW
NPAIRSTEPS = BPW // 2

CHUNK_ROWS = (64, 64, 64, 64, 64, 5)
CHUNK_OFF = tuple(sum(CHUNK_ROWS[:c]) for c in range(len(CHUNK_ROWS)))
NCHUNK = len(CHUNK_ROWS)


def _chunk_segments():
    bounds = list(CHUNK_OFF) + [N_PAIRS]
    segs = [[] for _ in range(NCHUNK)]
    row = 0
    for i in range(N_FIELDS - 1):
        j = i + 1
        while j < N_FIELDS:
            c = max(cc for cc in range(NCHUNK) if bounds[cc] <= row)
            take = min(N_FIELDS - j, bounds[c + 1] - row)
            segs[c].append((i, j, j + take, row))
            j += take
            row += take
    return segs


_SEGS = _chunk_segments()


def _compute_chunk(srcs, dsts, cidx):
    for i, j_lo, j_hi, dst_row in _SEGS[cidx]:
        aa = [[s[i, pl.ds(v * L, L)] for v in range(NV)] for s in srcs]
        if j_hi - j_lo <= 2:
            for j in range(j_lo, j_hi):
                for s, d, a in zip(srcs, dsts, aa):
                    for v in range(NV):
                        d[dst_row + j - j_lo, pl.ds(v * L, L)] = (
                            a[v] * s[j, pl.ds(v * L, L)])
        else:
            @plsc.parallel_loop(j_lo, j_hi, unroll=2)
            def _j(j, aa=aa, j_lo=j_lo, dst_row=dst_row):
                r = dst_row + j - j_lo
                for s, d, a in zip(srcs, dsts, aa):
                    for v in range(NV):
                        d[r, pl.ds(v * L, L)] = a[v] * s[j, pl.ds(v * L, L)]


def _sc_body(in_hbm, out_hbm, in_v, ring, sem_i0, sem_i1, sem_o0, sem_o1):
    wid = lax.axis_index("s") * NC + lax.axis_index("c")
    base = wid * BPW
    in_sems = (sem_i0, sem_i1)
    out_sems = (sem_o0, sem_o1)

    def in_copies(t, p, sem):
        return [
            pltpu.make_async_copy(in_hbm.at[base + 2 * t + m],
                                  in_v.at[2 * p + m], sem)
            for m in range(2)
        ]

    def chunk_copy(m, cidx, b, sem):
        return pltpu.make_async_copy(
            ring.at[m, pl.ds(CHUNK_OFF[cidx], CHUNK_ROWS[cidx])],
            out_hbm.at[b, pl.ds(CHUNK_OFF[cidx], CHUNK_ROWS[cidx])],
            sem,
        )

    def drain_all():
        for m in range(2):
            for cidx in range(NCHUNK):
                chunk_copy(m, cidx, base, out_sems[m]).wait()

    for cc in in_copies(0, 0, in_sems[0]):
        cc.start()

    def step(tt, c):
        for p in range(2):
            t = 2 * tt + p

            @pl.when(t + 1 < NPAIRSTEPS)
            def _prefetch(t=t, p=p):
                for cc in in_copies(t + 1, 1 - p, in_sems[1 - p]):
                    cc.start()

            for cc in in_copies(t, p, in_sems[p]):
                cc.wait()

            @pl.when(t > 0)
            def _drain():
                drain_all()

            srcs = [in_v.at[2 * p], in_v.at[2 * p + 1]]
            dsts = [ring.at[0], ring.at[1]]
            for cidx in range(NCHUNK):
                _compute_chunk(srcs, dsts, cidx)
                for m in range(2):
                    chunk_copy(m, cidx, base + 2 * t + m, out_sems[m]).start()
        return c

    lax.fori_loop(0, NPAIRSTEPS // 2, step, 0)
    drain_all()


def kernel(inputs):
    return pl.kernel(
        _sc_body,
        out_type=jax.ShapeDtypeStruct((B, N_PAIRS, D), jnp.float32),
        mesh=plsc.VectorSubcoreMesh(core_axis_name="c", subcore_axis_name="s"),
        scratch_types=[
            pltpu.VMEM((4, N_FIELDS, D), jnp.float32),
            pltpu.VMEM((2, N_PAIRS, D), jnp.float32),
            pltpu.SemaphoreType.DMA,
            pltpu.SemaphoreType.DMA,
            pltpu.SemaphoreType.DMA,
            pltpu.SemaphoreType.DMA,
        ],
    )(inputs)


# speedup vs baseline: 1.0784x; 1.0784x over previous
import jax
import jax.numpy as jnp
from jax import lax
from jax.experimental import pallas as pl
from jax.experimental.pallas import tpu as pltpu
from jax.experimental.pallas import tpu_sc as plsc

N_FIELDS = 26
N_PAIRS = N_FIELDS * (N_FIELDS - 1) // 2
D = 128
L = 16
NV = D // L
B = 4096
NC = 2
NS = 16
NW = NC * NS
BPW = B // NW
NPAIRSTEPS = BPW // 2

CHUNK_ROWS = (64, 64, 64, 64, 64, 5)
CHUNK_OFF = tuple(sum(CHUNK_ROWS[:c]) for c in range(len(CHUNK_ROWS)))
NCHUNK = len(CHUNK_ROWS)


def _chunk_segments():
    bounds = list(CHUNK_OFF) + [N_PAIRS]
    segs = [[] for _ in range(NCHUNK)]
    row = 0
    for i in range(N_FIELDS - 1):
        j = i + 1
        while j < N_FIELDS:
            c = max(cc for cc in range(NCHUNK) if bounds[cc] <= row)
            take = min(N_FIELDS - j, bounds[c + 1] - row)
            segs[c].append((i, j, j + take, row))
            j += take
            row += take
    return segs


_SEGS = _chunk_segments()


def _compute_chunk(srcs, dsts, cidx):
    for i, j_lo, j_hi, dst_row in _SEGS[cidx]:
        aa = [[s[i, pl.ds(v * L, L)] for v in range(NV)] for s in srcs]
        if j_hi - j_lo <= 2:
            for j in range(j_lo, j_hi):
                for s, d, a in zip(srcs, dsts, aa):
                    for v in range(NV):
                        d[dst_row + j - j_lo, pl.ds(v * L, L)] = (
                            a[v] * s[j, pl.ds(v * L, L)])
        else:
            @plsc.parallel_loop(j_lo, j_hi, unroll=1)
            def _j(j, aa=aa, j_lo=j_lo, dst_row=dst_row):
                r = dst_row + j - j_lo
                for s, d, a in zip(srcs, dsts, aa):
                    for v in range(NV):
                        d[r, pl.ds(v * L, L)] = a[v] * s[j, pl.ds(v * L, L)]


def _sc_body(in_hbm, out_hbm, in_v, ring, sem_i0, sem_i1, sem_o0, sem_o1):
    wid = lax.axis_index("s") * NC + lax.axis_index("c")
    base = wid * BPW
    in_sems = (sem_i0, sem_i1)
    out_sems = (sem_o0, sem_o1)

    def in_copies(t, p, sem):
        return [
            pltpu.make_async_copy(in_hbm.at[base + 2 * t + m],
                                  in_v.at[2 * p + m], sem)
            for m in range(2)
        ]

    def chunk_copy(m, cidx, b, sem):
        return pltpu.make_async_copy(
            ring.at[m, pl.ds(CHUNK_OFF[cidx], CHUNK_ROWS[cidx])],
            out_hbm.at[b, pl.ds(CHUNK_OFF[cidx], CHUNK_ROWS[cidx])],
            sem,
        )

    def drain_all():
        for m in range(2):
            for cidx in range(NCHUNK):
                chunk_copy(m, cidx, base, out_sems[m]).wait()

    for cc in in_copies(0, 0, in_sems[0]):
        cc.start()

    def step(tt, c):
        for p in range(2):
            t = 2 * tt + p

            @pl.when(t + 1 < NPAIRSTEPS)
            def _prefetch(t=t, p=p):
                for cc in in_copies(t + 1, 1 - p, in_sems[1 - p]):
                    cc.start()

            for cc in in_copies(t, p, in_sems[p]):
                cc.wait()

            @pl.when(t > 0)
            def _drain():
                drain_all()

            srcs = [in_v.at[2 * p], in_v.at[2 * p + 1]]
            dsts = [ring.at[0], ring.at[1]]
            for cidx in range(NCHUNK):
                _compute_chunk(srcs, dsts, cidx)
                for m in range(2):
                    chunk_copy(m, cidx, base + 2 * t + m, out_sems[m]).start()
        return c

    lax.fori_loop(0, NPAIRSTEPS // 2, step, 0)
    drain_all()


def kernel(inputs):
    return pl.kernel(
        _sc_body,
        out_type=jax.ShapeDtypeStruct((B, N_PAIRS, D), jnp.float32),
        mesh=plsc.VectorSubcoreMesh(core_axis_name="c", subcore_axis_name="s"),
        scratch_types=[
            pltpu.VMEM((4, N_FIELDS, D), jnp.float32),
            pltpu.VMEM((2, N_PAIRS, D), jnp.float32),
            pltpu.SemaphoreType.DMA,
            pltpu.SemaphoreType.DMA,
            pltpu.SemaphoreType.DMA,
            pltpu.SemaphoreType.DMA,
        ],
    )(inputs)


# speedup vs baseline: 1.2486x; 1.1578x over previous
import jax
import jax.numpy as jnp
from jax.experimental import pallas as pl
from jax.experimental.pallas import tpu as pltpu

N_FIELDS = 26
N_PAIRS = N_FIELDS * (N_FIELDS - 1) // 2
BB = 128


def _pair_body(in_ref, out_ref):
    x = in_ref[...]
    off = 0
    for i in range(N_FIELDS - 1):
        w = N_FIELDS - 1 - i
        out_ref[:, off:off + w, :] = x[:, i:i + 1, :] * x[:, i + 1:, :]
        off += w


def kernel(inputs):
    b, f, d = inputs.shape
    grid = (b // BB,)
    return pl.pallas_call(
        _pair_body,
        grid=grid,
        in_specs=[pl.BlockSpec((BB, f, d), lambda g: (g, 0, 0))],
        out_specs=pl.BlockSpec((BB, N_PAIRS, d), lambda g: (g, 0, 0)),
        out_shape=jax.ShapeDtypeStruct((b, N_PAIRS, d), jnp.float32),
    )(inputs)
